# Initial kernel scaffold; baseline (speedup 1.0000x reference)
import jax
import jax.numpy as jnp
from jax.experimental import pallas as pl


def kernel(similarity_scores, positive_mask):
    raise NotImplementedError("write your pallas kernel here")


# speedup vs baseline: 2.3959x; 2.3959x over previous
import functools
import math

import jax
import jax.numpy as jnp
from jax.experimental import pallas as pl

_K = 10


def _loss_kernel(scores_ref, mask_ref, out_ref):
    r = scores_ref.shape[0]
    n = scores_ref.shape[1]
    b = jax.lax.bitcast_convert_type(scores_ref[...], jnp.int32)
    key = jnp.where(b >= 0, b, b ^ jnp.int32(0x7FFFFFFF))
    label_bit = mask_ref[...].astype(jnp.int32)
    key = (key & jnp.int32(-2)) | label_bit

    iota = jax.lax.broadcasted_iota(jnp.int32, (r, n), 1)

    actual = jnp.zeros((r,), jnp.float32)
    mpos = jnp.zeros((r,), jnp.float32)
    weights = [1.0 / math.log2(kk + 2.0) for kk in range(_K)]
    for kk in range(_K):
        m = jnp.max(key, axis=1)
        eq = key == m[:, None]
        idx = jnp.min(jnp.where(eq, iota, jnp.int32(n)), axis=1)
        sel = iota == idx[:, None]
        lab = (m & 1).astype(jnp.float32)
        actual = actual + lab * jnp.float32(weights[kk])
        mpos = mpos + lab
        key = jnp.where(sel, jnp.int32(-(2**31)), key)

    ideal = jnp.zeros((r,), jnp.float32)
    for kk in range(_K):
        ideal = ideal + jnp.where(mpos > kk, jnp.float32(weights[kk]), 0.0)
    per_query = jnp.where(ideal > 0.0, 1.0 - actual / jnp.maximum(ideal, 1e-30), 0.0)
    block_sum = jnp.sum(per_query)

    @pl.when(pl.program_id(0) == 0)
    def _init():
        out_ref[...] = jnp.zeros((1, 1), jnp.float32)

    out_ref[...] += jnp.full((1, 1), 1.0, jnp.float32) * block_sum


@functools.partial(jax.jit, static_argnames=("block_rows",))
def _run(scores, mask, block_rows=128):
    bq, n = scores.shape
    grid = (bq // block_rows,)
    total = pl.pallas_call(
        _loss_kernel,
        grid=grid,
        in_specs=[
            pl.BlockSpec((block_rows, n), lambda i: (i, 0)),
            pl.BlockSpec((block_rows, n), lambda i: (i, 0)),
        ],
        out_specs=pl.BlockSpec((1, 1), lambda i: (0, 0)),
        out_shape=jax.ShapeDtypeStruct((1, 1), jnp.float32),
    )(scores, mask)
    return total[0, 0] / jnp.float32(bq)


def kernel(similarity_scores, positive_mask):
    return _run(similarity_scores, positive_mask.astype(jnp.int8))
